# Initial kernel scaffold; baseline (speedup 1.0000x reference)
import jax
import jax.numpy as jnp
from jax.experimental import pallas as pl


def kernel(x, pos, edge_index, batch, embeddings, w_rbf, w_msg, w_upd, w_att, w_out):
    raise NotImplementedError("write your pallas kernel here")


# speedup vs baseline: 2.4105x; 2.4105x over previous
import functools

import jax
import jax.numpy as jnp
from jax import lax
from jax.experimental import pallas as pl
from jax.experimental.pallas import tpu as pltpu
from jax.experimental.pallas import tpu_sc as plsc

N = 10000
E = 320000
DIM = 128
NUM_RBF = 16
N_LAYER = 3
CUTOFF = 5.0
NUM_GRAPHS = 512

NC = 2
NS = 16
NW = NC * NS
C = 128
NCH = 80
E_PAD = NW * NCH * C
ROWS_PER_TILE = 640
AGG_ROWS = NS * ROWS_PER_TILE
POS_PAD = 30720

NP = 10240
NB = 2048
EB = 4096


def _dist_body(pos_hbm, j_hbm, i_hbm, d2_hbm, posv, jv, iv, d2v):
    w = lax.axis_index("s") * NC + lax.axis_index("c")
    pltpu.sync_copy(pos_hbm, posv)
    pltpu.sync_copy(j_hbm.at[pl.ds(w * NCH, NCH)], jv)
    pltpu.sync_copy(i_hbm.at[pl.ds(w * NCH, NCH)], iv)

    def chunk(c, _):
        def vec(v, _):
            sl = pl.ds(v * 16, 16)
            jj = jv[c, sl] * 3
            ii = iv[c, sl] * 3
            xj = plsc.load_gather(posv, [jj])
            yj = plsc.load_gather(posv, [jj + 1])
            zj = plsc.load_gather(posv, [jj + 2])
            xi = plsc.load_gather(posv, [ii])
            yi = plsc.load_gather(posv, [ii + 1])
            zi = plsc.load_gather(posv, [ii + 2])
            dx = xi - xj
            dy = yi - yj
            dz = zi - zj
            d2v[c, sl] = dx * dx + dy * dy + dz * dz
            return 0

        lax.fori_loop(0, C // 16, vec, 0)
        return 0

    lax.fori_loop(0, NCH, chunk, 0)
    pltpu.sync_copy(d2v, d2_hbm.at[pl.ds(w * NCH, NCH)])


def _dist_call(pos_flat, j2, i2):
    mesh = plsc.VectorSubcoreMesh(core_axis_name="c", subcore_axis_name="s")
    return pl.kernel(
        _dist_body,
        out_type=jax.ShapeDtypeStruct((NW * NCH, C), jnp.float32),
        mesh=mesh,
        compiler_params=pltpu.CompilerParams(needs_layout_passes=False),
        scratch_types=[
            pltpu.VMEM((POS_PAD,), jnp.float32),
            pltpu.VMEM((NCH, C), jnp.int32),
            pltpu.VMEM((NCH, C), jnp.int32),
            pltpu.VMEM((NCH, C), jnp.float32),
        ],
    )(pos_flat, j2, i2)


def _agg_body(hw_hbm, ea_hbm, j_hbm, i_hbm, out_hbm, jv8, iv8, rows, eav, aggs, sem):
    c_id = lax.axis_index("c")
    s_id = lax.axis_index("s")
    w = s_id * NC + c_id

    def zrow(r, _):
        for d in range(8):
            rows[r, pl.ds(d * 16, 16)] = jnp.zeros((16,), jnp.float32)
        return 0

    lax.fori_loop(0, C, zrow, 0)
    for z in range(ROWS_PER_TILE // C):
        pltpu.sync_copy(rows, aggs.at[pl.ds(s_id * ROWS_PER_TILE + z * C, C)])
    plsc.subcore_barrier()

    def group(g, _):
        pltpu.sync_copy(j_hbm.at[pl.ds(w * NCH + g * 8, 8)], jv8)
        pltpu.sync_copy(i_hbm.at[pl.ds(w * NCH + g * 8, 8)], iv8)
        for k in range(8):
            base = (w * NCH + g * 8 + k) * C
            pltpu.sync_copy(ea_hbm.at[pl.ds(base, C)], eav)
            pltpu.async_copy(hw_hbm.at[jv8.at[k]], rows, sem).wait()

            def mulrow(r, _):
                for d in range(8):
                    sl = pl.ds(d * 16, 16)
                    rows[r, sl] = rows[r, sl] * eav[r, sl]
                return 0

            lax.fori_loop(0, C, mulrow, 0)
            pltpu.sync_copy(rows, aggs.at[iv8.at[k]], add=True)
        return 0

    lax.fori_loop(0, NCH // 8, group, 0)
    plsc.subcore_barrier()
    pltpu.sync_copy(
        aggs.at[pl.ds(s_id * ROWS_PER_TILE, ROWS_PER_TILE)],
        out_hbm.at[c_id, pl.ds(s_id * ROWS_PER_TILE, ROWS_PER_TILE)],
    )


def _agg_call(hw, ea, j2, i2):
    mesh = plsc.VectorSubcoreMesh(core_axis_name="c", subcore_axis_name="s")
    return pl.kernel(
        _agg_body,
        out_type=jax.ShapeDtypeStruct((NC, AGG_ROWS, DIM), jnp.float32),
        mesh=mesh,
        compiler_params=pltpu.CompilerParams(needs_layout_passes=False),
        scratch_types=[
            pltpu.VMEM((8, C), jnp.int32),
            pltpu.VMEM((8, C), jnp.int32),
            pltpu.VMEM((C, DIM), jnp.float32),
            pltpu.VMEM((C, DIM), jnp.float32),
            pltpu.VMEM_SHARED((AGG_ROWS, DIM), jnp.float32),
            pltpu.SemaphoreType.DMA,
        ],
    )(hw, ea, j2, i2)


def _embed_body(x_ref, emb_ref, wmsg_ref, h_ref, hw_ref):
    xb = x_ref[...]
    onehot = (xb[:, None] == lax.broadcasted_iota(jnp.int32, (NB, 32), 1))
    h = jnp.dot(onehot.astype(jnp.float32), emb_ref[...],
                preferred_element_type=jnp.float32)
    h_ref[...] = h
    hw_ref[...] = jnp.dot(h, wmsg_ref[...], preferred_element_type=jnp.float32)


def _embed_call(x, emb_pad, wmsg0):
    return pl.pallas_call(
        _embed_body,
        grid=(NP // NB,),
        in_specs=[
            pl.BlockSpec((NB,), lambda b: (b,)),
            pl.BlockSpec((32, DIM), lambda b: (0, 0)),
            pl.BlockSpec((DIM, DIM), lambda b: (0, 0)),
        ],
        out_specs=[
            pl.BlockSpec((NB, DIM), lambda b: (b, 0)),
            pl.BlockSpec((NB, DIM), lambda b: (b, 0)),
        ],
        out_shape=[
            jax.ShapeDtypeStruct((NP, DIM), jnp.float32),
            jax.ShapeDtypeStruct((NP, DIM), jnp.float32),
        ],
    )(x, emb_pad, wmsg0)


def _ea_body(d2_ref, npad_ref, wrbf_ref, ea_ref):
    d2 = d2_ref[...]
    dist = jnp.sqrt(d2 + 1e-12)
    npad = npad_ref[...]
    arg = npad * (jnp.pi / CUTOFF) * dist[:, None]
    rbf = jnp.sin(arg) / (dist[:, None] + 1e-6)
    env = jnp.where(dist < CUTOFF, (1.0 - dist / CUTOFF) ** 2, 0.0)
    rbf = rbf * env[:, None]
    ea_ref[...] = jnp.dot(rbf, wrbf_ref[...], preferred_element_type=jnp.float32)


def _ea_call(d2_flat, npad, wrbf_pad):
    return pl.pallas_call(
        _ea_body,
        grid=(E_PAD // EB,),
        in_specs=[
            pl.BlockSpec((EB,), lambda b: (b,)),
            pl.BlockSpec((1, DIM), lambda b: (0, 0)),
            pl.BlockSpec((DIM, DIM), lambda b: (0, 0)),
        ],
        out_specs=pl.BlockSpec((EB, DIM), lambda b: (b, 0)),
        out_shape=jax.ShapeDtypeStruct((E_PAD, DIM), jnp.float32),
    )(d2_flat, npad, wrbf_pad)


def _upd_body(h_ref, a0_ref, a1_ref, wupd_ref, wmsgn_ref, wout_ref, watt_ref,
              hn_ref, hw_ref, o_ref, a_ref):
    agg = a0_ref[...] + a1_ref[...]
    u = jnp.dot(agg, wupd_ref[...], preferred_element_type=jnp.float32)
    hn = h_ref[...] + u * jax.nn.sigmoid(u)
    hn_ref[...] = hn
    hw_ref[...] = jnp.dot(hn, wmsgn_ref[...], preferred_element_type=jnp.float32)
    o_ref[...] = jnp.sum(hn * wout_ref[...], axis=1)
    a_ref[...] = jnp.sum(hn * watt_ref[...], axis=1)


def _upd_call(h, a0, a1, wupd, wmsgn, wout_row, watt_row):
    return pl.pallas_call(
        _upd_body,
        grid=(NP // NB,),
        in_specs=[
            pl.BlockSpec((NB, DIM), lambda b: (b, 0)),
            pl.BlockSpec((NB, DIM), lambda b: (b, 0)),
            pl.BlockSpec((NB, DIM), lambda b: (b, 0)),
            pl.BlockSpec((DIM, DIM), lambda b: (0, 0)),
            pl.BlockSpec((DIM, DIM), lambda b: (0, 0)),
            pl.BlockSpec((1, DIM), lambda b: (0, 0)),
            pl.BlockSpec((1, DIM), lambda b: (0, 0)),
        ],
        out_specs=[
            pl.BlockSpec((NB, DIM), lambda b: (b, 0)),
            pl.BlockSpec((NB, DIM), lambda b: (b, 0)),
            pl.BlockSpec((NB,), lambda b: (b,)),
            pl.BlockSpec((NB,), lambda b: (b,)),
        ],
        out_shape=[
            jax.ShapeDtypeStruct((NP, DIM), jnp.float32),
            jax.ShapeDtypeStruct((NP, DIM), jnp.float32),
            jax.ShapeDtypeStruct((NP,), jnp.float32),
            jax.ShapeDtypeStruct((NP,), jnp.float32),
        ],
    )(h, a0, a1, wupd, wmsgn, wout_row, watt_row)


def _pool_body(o0_ref, o1_ref, o2_ref, a0_ref, a1_ref, a2_ref, h_ref, b_ref,
               feat_ref, misc_ref, feat_acc, misc_acc):
    b = pl.program_id(0)

    @pl.when(b == 0)
    def _():
        feat_acc[...] = jnp.zeros((NUM_GRAPHS, DIM), jnp.float32)
        misc_acc[...] = jnp.zeros((NUM_GRAPHS, DIM), jnp.float32)

    def leaky(v):
        return jnp.where(v > 0, v, 0.2 * v)

    l0 = leaky(a0_ref[...])
    l1 = leaky(a1_ref[...])
    l2 = leaky(a2_ref[...])
    m = jnp.maximum(jnp.maximum(l0, l1), l2)
    e0 = jnp.exp(l0 - m)
    e1 = jnp.exp(l1 - m)
    e2 = jnp.exp(l2 - m)
    s = e0 + e1 + e2
    energy = (o0_ref[...] * e0 + o1_ref[...] * e1 + o2_ref[...] * e2) / s

    batch = b_ref[...]
    onehot = (lax.broadcasted_iota(jnp.int32, (NUM_GRAPHS, NB), 0)
              == batch[None, :]).astype(jnp.float32)
    feat_acc[...] += jnp.dot(onehot, h_ref[...], preferred_element_type=jnp.float32)
    lane = lax.broadcasted_iota(jnp.int32, (NB, DIM), 1)
    eo = jnp.where(lane == 0, energy[:, None], 0.0) + jnp.where(lane == 1, 1.0, 0.0)
    misc_acc[...] += jnp.dot(onehot, eo, preferred_element_type=jnp.float32)

    @pl.when(b == NP // NB - 1)
    def _():
        misc = misc_acc[...]
        lane2 = lax.broadcasted_iota(jnp.int32, (NUM_GRAPHS, DIM), 1)
        counts = jnp.sum(jnp.where(lane2 == 1, misc, 0.0), axis=1, keepdims=True)
        feat_ref[...] = feat_acc[...] / jnp.maximum(counts, 1.0)
        misc_ref[...] = misc


def _pool_call(o0, o1, o2, a0, a1, a2, h, batch):
    return pl.pallas_call(
        _pool_body,
        grid=(NP // NB,),
        in_specs=[pl.BlockSpec((NB,), lambda b: (b,))] * 6
        + [
            pl.BlockSpec((NB, DIM), lambda b: (b, 0)),
            pl.BlockSpec((NB,), lambda b: (b,)),
        ],
        out_specs=[
            pl.BlockSpec((NUM_GRAPHS, DIM), lambda b: (0, 0)),
            pl.BlockSpec((NUM_GRAPHS, DIM), lambda b: (0, 0)),
        ],
        out_shape=[
            jax.ShapeDtypeStruct((NUM_GRAPHS, DIM), jnp.float32),
            jax.ShapeDtypeStruct((NUM_GRAPHS, DIM), jnp.float32),
        ],
        scratch_shapes=[
            pltpu.VMEM((NUM_GRAPHS, DIM), jnp.float32),
            pltpu.VMEM((NUM_GRAPHS, DIM), jnp.float32),
        ],
    )(o0, o1, o2, a0, a1, a2, h, batch)


def kernel(x, pos, edge_index, batch, embeddings, w_rbf, w_msg, w_upd, w_att, w_out):
    pad = E_PAD - E
    j = jnp.concatenate([edge_index[0], jnp.zeros((pad,), jnp.int32)])
    i = jnp.concatenate([edge_index[1], jnp.full((pad,), N, jnp.int32)])
    j2 = j.reshape(NW * NCH, C)
    i2 = i.reshape(NW * NCH, C)
    pos_flat = jnp.concatenate(
        [pos.reshape(-1), jnp.zeros((POS_PAD - 3 * N,), jnp.float32)])
    emb_pad = jnp.pad(embeddings, ((0, 32 - embeddings.shape[0]), (0, 0)))
    npad = jnp.pad(jnp.arange(1, NUM_RBF + 1, dtype=jnp.float32),
                   (0, DIM - NUM_RBF)).reshape(1, DIM)
    wrbf_pad = jnp.pad(w_rbf, ((0, DIM - NUM_RBF), (0, 0)))

    x_pad = jnp.pad(x, (0, NP - N))
    batch_pad = jnp.pad(batch, (0, NP - N), constant_values=NUM_GRAPHS)
    h, hw = _embed_call(x_pad, emb_pad, w_msg[0])
    d2 = _dist_call(pos_flat, j2, i2)
    ea = _ea_call(d2.reshape(-1), npad, wrbf_pad)

    outs, atts = [], []
    for l in range(N_LAYER):
        agg2 = _agg_call(hw, ea, j2, i2)
        wmsgn = w_msg[l + 1] if l + 1 < N_LAYER else w_msg[0]
        h, hw, o_l, a_l = _upd_call(
            h, agg2[0], agg2[1], w_upd[l], wmsgn,
            w_out[l].reshape(1, DIM), w_att[l].reshape(1, DIM))
        outs.append(o_l)
        atts.append(a_l)

    feat, misc = _pool_call(outs[0], outs[1], outs[2],
                            atts[0], atts[1], atts[2], h, batch_pad)
    return misc[:, 0], h[:N], feat


# speedup vs baseline: 2.8726x; 1.1917x over previous
import functools

import jax
import jax.numpy as jnp
from jax import lax
from jax.experimental import pallas as pl
from jax.experimental.pallas import tpu as pltpu
from jax.experimental.pallas import tpu_sc as plsc

N = 10000
E = 320000
DIM = 128
NUM_RBF = 16
N_LAYER = 3
CUTOFF = 5.0
NUM_GRAPHS = 512

NC = 2
NS = 16
NW = NC * NS
C = 128
NCH = 80
E_PAD = NW * NCH * C
ROWS_PER_TILE = 640
AGG_ROWS = NS * ROWS_PER_TILE
POS_PAD = 30720

NP = 10240
NB = 2048
EB = 4096


def _dist_body(pos_hbm, j_hbm, i_hbm, d2_hbm, posv, jv, iv, d2v):
    w = lax.axis_index("s") * NC + lax.axis_index("c")
    pltpu.sync_copy(pos_hbm, posv)
    pltpu.sync_copy(j_hbm.at[pl.ds(w * NCH, NCH)], jv)
    pltpu.sync_copy(i_hbm.at[pl.ds(w * NCH, NCH)], iv)

    def chunk(c, _):
        def vec(v, _):
            sl = pl.ds(v * 16, 16)
            jj = jv[c, sl] * 3
            ii = iv[c, sl] * 3
            xj = plsc.load_gather(posv, [jj])
            yj = plsc.load_gather(posv, [jj + 1])
            zj = plsc.load_gather(posv, [jj + 2])
            xi = plsc.load_gather(posv, [ii])
            yi = plsc.load_gather(posv, [ii + 1])
            zi = plsc.load_gather(posv, [ii + 2])
            dx = xi - xj
            dy = yi - yj
            dz = zi - zj
            d2v[c, sl] = dx * dx + dy * dy + dz * dz
            return 0

        lax.fori_loop(0, C // 16, vec, 0)
        return 0

    lax.fori_loop(0, NCH, chunk, 0)
    pltpu.sync_copy(d2v, d2_hbm.at[pl.ds(w * NCH, NCH)])


def _dist_call(pos_flat, j2, i2):
    mesh = plsc.VectorSubcoreMesh(core_axis_name="c", subcore_axis_name="s")
    return pl.kernel(
        _dist_body,
        out_type=jax.ShapeDtypeStruct((NW * NCH, C), jnp.float32),
        mesh=mesh,
        compiler_params=pltpu.CompilerParams(needs_layout_passes=False),
        scratch_types=[
            pltpu.VMEM((POS_PAD,), jnp.float32),
            pltpu.VMEM((NCH, C), jnp.int32),
            pltpu.VMEM((NCH, C), jnp.int32),
            pltpu.VMEM((NCH, C), jnp.float32),
        ],
    )(pos_flat, j2, i2)


CB = 64
NCH2 = E_PAD // (NW * CB)
GRP = 8
NGRP = NCH2 // GRP


def _agg_body(hw_hbm, ea_hbm, j_hbm, i_hbm, out_hbm,
              jv, iv, r0, r1, e0, e1, aggs, sg0, sg1, se0, se1, ss0, ss1):
    c_id = lax.axis_index("c")
    s_id = lax.axis_index("s")
    w = s_id * NC + c_id
    rbufs = (r0, r1)
    ebufs = (e0, e1)
    sgs = (sg0, sg1)
    ses = (se0, se1)
    sss = (ss0, ss1)

    def zrow(r, _):
        for d in range(8):
            r0[r, pl.ds(d * 16, 16)] = jnp.zeros((16,), jnp.float32)
        return 0

    lax.fori_loop(0, CB, zrow, 0)
    for z in range(ROWS_PER_TILE // CB):
        pltpu.sync_copy(r0, aggs.at[pl.ds(s_id * ROWS_PER_TILE + z * CB, CB)])
    plsc.subcore_barrier()

    def group(g, _):
        gbase = w * NCH2 + g * GRP
        pltpu.sync_copy(j_hbm.at[pl.ds(gbase, GRP)], jv)
        pltpu.sync_copy(i_hbm.at[pl.ds(gbase, GRP)], iv)
        d_g = [None, None]
        d_e = [None, None]
        d_s = [None, None]
        d_g[0] = pltpu.async_copy(hw_hbm.at[jv.at[0]], r0, sg0)
        d_e[0] = pltpu.async_copy(ea_hbm.at[pl.ds(gbase * CB, CB)], e0, se0)
        for k in range(GRP):
            kb = k % 2
            nb = (k + 1) % 2
            if k < GRP - 1:
                if k >= 1:
                    d_s[nb].wait()
                d_g[nb] = pltpu.async_copy(hw_hbm.at[jv.at[k + 1]], rbufs[nb], sgs[nb])
                d_e[nb] = pltpu.async_copy(
                    ea_hbm.at[pl.ds((gbase + k + 1) * CB, CB)], ebufs[nb], ses[nb])
            d_g[kb].wait()
            d_e[kb].wait()
            rows = rbufs[kb]
            eav = ebufs[kb]

            def mulrow(r, _):
                for d in range(8):
                    sl = pl.ds(d * 16, 16)
                    rows[r, sl] = rows[r, sl] * eav[r, sl]
                return 0

            lax.fori_loop(0, CB, mulrow, 0)
            d_s[kb] = pltpu.async_copy(rows, aggs.at[iv.at[k]], sss[kb], add=True)
        d_s[0].wait()
        d_s[1].wait()
        return 0

    lax.fori_loop(0, NGRP, group, 0)
    plsc.subcore_barrier()
    pltpu.sync_copy(
        aggs.at[pl.ds(s_id * ROWS_PER_TILE, ROWS_PER_TILE)],
        out_hbm.at[c_id, pl.ds(s_id * ROWS_PER_TILE, ROWS_PER_TILE)],
    )


def _agg_call(hw, ea, j64, i64):
    mesh = plsc.VectorSubcoreMesh(core_axis_name="c", subcore_axis_name="s")
    return pl.kernel(
        _agg_body,
        out_type=jax.ShapeDtypeStruct((NC, AGG_ROWS, DIM), jnp.float32),
        mesh=mesh,
        compiler_params=pltpu.CompilerParams(needs_layout_passes=False),
        scratch_types=[
            pltpu.VMEM((GRP, CB), jnp.int32),
            pltpu.VMEM((GRP, CB), jnp.int32),
            pltpu.VMEM((CB, DIM), jnp.float32),
            pltpu.VMEM((CB, DIM), jnp.float32),
            pltpu.VMEM((CB, DIM), jnp.float32),
            pltpu.VMEM((CB, DIM), jnp.float32),
            pltpu.VMEM_SHARED((AGG_ROWS, DIM), jnp.float32),
            pltpu.SemaphoreType.DMA,
            pltpu.SemaphoreType.DMA,
            pltpu.SemaphoreType.DMA,
            pltpu.SemaphoreType.DMA,
            pltpu.SemaphoreType.DMA,
            pltpu.SemaphoreType.DMA,
        ],
    )(hw, ea, j64, i64)


def _embed_body(x_ref, emb_ref, wmsg_ref, h_ref, hw_ref):
    xb = x_ref[...]
    onehot = (xb[:, None] == lax.broadcasted_iota(jnp.int32, (NB, 32), 1))
    h = jnp.dot(onehot.astype(jnp.float32), emb_ref[...],
                preferred_element_type=jnp.float32)
    h_ref[...] = h
    hw_ref[...] = jnp.dot(h, wmsg_ref[...], preferred_element_type=jnp.float32)


def _embed_call(x, emb_pad, wmsg0):
    return pl.pallas_call(
        _embed_body,
        grid=(NP // NB,),
        in_specs=[
            pl.BlockSpec((NB,), lambda b: (b,)),
            pl.BlockSpec((32, DIM), lambda b: (0, 0)),
            pl.BlockSpec((DIM, DIM), lambda b: (0, 0)),
        ],
        out_specs=[
            pl.BlockSpec((NB, DIM), lambda b: (b, 0)),
            pl.BlockSpec((NB, DIM), lambda b: (b, 0)),
        ],
        out_shape=[
            jax.ShapeDtypeStruct((NP, DIM), jnp.float32),
            jax.ShapeDtypeStruct((NP, DIM), jnp.float32),
        ],
    )(x, emb_pad, wmsg0)


def _ea_body(d2_ref, npad_ref, wrbf_ref, ea_ref):
    d2 = d2_ref[...]
    dist = jnp.sqrt(d2 + 1e-12)
    npad = npad_ref[...]
    arg = npad * (jnp.pi / CUTOFF) * dist[:, None]
    rbf = jnp.sin(arg) / (dist[:, None] + 1e-6)
    env = jnp.where(dist < CUTOFF, (1.0 - dist / CUTOFF) ** 2, 0.0)
    rbf = rbf * env[:, None]
    ea_ref[...] = jnp.dot(rbf, wrbf_ref[...], preferred_element_type=jnp.float32)


def _ea_call(d2_flat, npad, wrbf_pad):
    return pl.pallas_call(
        _ea_body,
        grid=(E_PAD // EB,),
        in_specs=[
            pl.BlockSpec((EB,), lambda b: (b,)),
            pl.BlockSpec((1, DIM), lambda b: (0, 0)),
            pl.BlockSpec((DIM, DIM), lambda b: (0, 0)),
        ],
        out_specs=pl.BlockSpec((EB, DIM), lambda b: (b, 0)),
        out_shape=jax.ShapeDtypeStruct((E_PAD, DIM), jnp.float32),
    )(d2_flat, npad, wrbf_pad)


def _upd_body(h_ref, a0_ref, a1_ref, wupd_ref, wmsgn_ref, wout_ref, watt_ref,
              hn_ref, hw_ref, o_ref, a_ref):
    agg = a0_ref[...] + a1_ref[...]
    u = jnp.dot(agg, wupd_ref[...], preferred_element_type=jnp.float32)
    hn = h_ref[...] + u * jax.nn.sigmoid(u)
    hn_ref[...] = hn
    hw_ref[...] = jnp.dot(hn, wmsgn_ref[...], preferred_element_type=jnp.float32)
    o_ref[...] = jnp.sum(hn * wout_ref[...], axis=1)
    a_ref[...] = jnp.sum(hn * watt_ref[...], axis=1)


def _upd_call(h, a0, a1, wupd, wmsgn, wout_row, watt_row):
    return pl.pallas_call(
        _upd_body,
        grid=(NP // NB,),
        in_specs=[
            pl.BlockSpec((NB, DIM), lambda b: (b, 0)),
            pl.BlockSpec((NB, DIM), lambda b: (b, 0)),
            pl.BlockSpec((NB, DIM), lambda b: (b, 0)),
            pl.BlockSpec((DIM, DIM), lambda b: (0, 0)),
            pl.BlockSpec((DIM, DIM), lambda b: (0, 0)),
            pl.BlockSpec((1, DIM), lambda b: (0, 0)),
            pl.BlockSpec((1, DIM), lambda b: (0, 0)),
        ],
        out_specs=[
            pl.BlockSpec((NB, DIM), lambda b: (b, 0)),
            pl.BlockSpec((NB, DIM), lambda b: (b, 0)),
            pl.BlockSpec((NB,), lambda b: (b,)),
            pl.BlockSpec((NB,), lambda b: (b,)),
        ],
        out_shape=[
            jax.ShapeDtypeStruct((NP, DIM), jnp.float32),
            jax.ShapeDtypeStruct((NP, DIM), jnp.float32),
            jax.ShapeDtypeStruct((NP,), jnp.float32),
            jax.ShapeDtypeStruct((NP,), jnp.float32),
        ],
    )(h, a0, a1, wupd, wmsgn, wout_row, watt_row)


def _pool_body(o0_ref, o1_ref, o2_ref, a0_ref, a1_ref, a2_ref, h_ref, b_ref,
               feat_ref, misc_ref, feat_acc, misc_acc):
    b = pl.program_id(0)

    @pl.when(b == 0)
    def _():
        feat_acc[...] = jnp.zeros((NUM_GRAPHS, DIM), jnp.float32)
        misc_acc[...] = jnp.zeros((NUM_GRAPHS, DIM), jnp.float32)

    def leaky(v):
        return jnp.where(v > 0, v, 0.2 * v)

    l0 = leaky(a0_ref[...])
    l1 = leaky(a1_ref[...])
    l2 = leaky(a2_ref[...])
    m = jnp.maximum(jnp.maximum(l0, l1), l2)
    e0 = jnp.exp(l0 - m)
    e1 = jnp.exp(l1 - m)
    e2 = jnp.exp(l2 - m)
    s = e0 + e1 + e2
    energy = (o0_ref[...] * e0 + o1_ref[...] * e1 + o2_ref[...] * e2) / s

    batch = b_ref[...]
    onehot = (lax.broadcasted_iota(jnp.int32, (NUM_GRAPHS, NB), 0)
              == batch[None, :]).astype(jnp.float32)
    feat_acc[...] += jnp.dot(onehot, h_ref[...], preferred_element_type=jnp.float32)
    lane = lax.broadcasted_iota(jnp.int32, (NB, DIM), 1)
    eo = jnp.where(lane == 0, energy[:, None], 0.0) + jnp.where(lane == 1, 1.0, 0.0)
    misc_acc[...] += jnp.dot(onehot, eo, preferred_element_type=jnp.float32)

    @pl.when(b == NP // NB - 1)
    def _():
        misc = misc_acc[...]
        lane2 = lax.broadcasted_iota(jnp.int32, (NUM_GRAPHS, DIM), 1)
        counts = jnp.sum(jnp.where(lane2 == 1, misc, 0.0), axis=1, keepdims=True)
        feat_ref[...] = feat_acc[...] / jnp.maximum(counts, 1.0)
        misc_ref[...] = misc


def _pool_call(o0, o1, o2, a0, a1, a2, h, batch):
    return pl.pallas_call(
        _pool_body,
        grid=(NP // NB,),
        in_specs=[pl.BlockSpec((NB,), lambda b: (b,))] * 6
        + [
            pl.BlockSpec((NB, DIM), lambda b: (b, 0)),
            pl.BlockSpec((NB,), lambda b: (b,)),
        ],
        out_specs=[
            pl.BlockSpec((NUM_GRAPHS, DIM), lambda b: (0, 0)),
            pl.BlockSpec((NUM_GRAPHS, DIM), lambda b: (0, 0)),
        ],
        out_shape=[
            jax.ShapeDtypeStruct((NUM_GRAPHS, DIM), jnp.float32),
            jax.ShapeDtypeStruct((NUM_GRAPHS, DIM), jnp.float32),
        ],
        scratch_shapes=[
            pltpu.VMEM((NUM_GRAPHS, DIM), jnp.float32),
            pltpu.VMEM((NUM_GRAPHS, DIM), jnp.float32),
        ],
    )(o0, o1, o2, a0, a1, a2, h, batch)


def kernel(x, pos, edge_index, batch, embeddings, w_rbf, w_msg, w_upd, w_att, w_out):
    pad = E_PAD - E
    j = jnp.concatenate([edge_index[0], jnp.zeros((pad,), jnp.int32)])
    i = jnp.concatenate([edge_index[1], jnp.full((pad,), N, jnp.int32)])
    j2 = j.reshape(NW * NCH, C)
    i2 = i.reshape(NW * NCH, C)
    j64 = j.reshape(E_PAD // CB, CB)
    i64 = i.reshape(E_PAD // CB, CB)
    pos_flat = jnp.concatenate(
        [pos.reshape(-1), jnp.zeros((POS_PAD - 3 * N,), jnp.float32)])
    emb_pad = jnp.pad(embeddings, ((0, 32 - embeddings.shape[0]), (0, 0)))
    npad = jnp.pad(jnp.arange(1, NUM_RBF + 1, dtype=jnp.float32),
                   (0, DIM - NUM_RBF)).reshape(1, DIM)
    wrbf_pad = jnp.pad(w_rbf, ((0, DIM - NUM_RBF), (0, 0)))

    x_pad = jnp.pad(x, (0, NP - N))
    batch_pad = jnp.pad(batch, (0, NP - N), constant_values=NUM_GRAPHS)
    h, hw = _embed_call(x_pad, emb_pad, w_msg[0])
    d2 = _dist_call(pos_flat, j2, i2)
    ea = _ea_call(d2.reshape(-1), npad, wrbf_pad)

    outs, atts = [], []
    for l in range(N_LAYER):
        agg2 = _agg_call(hw, ea, j64, i64)
        wmsgn = w_msg[l + 1] if l + 1 < N_LAYER else w_msg[0]
        h, hw, o_l, a_l = _upd_call(
            h, agg2[0], agg2[1], w_upd[l], wmsgn,
            w_out[l].reshape(1, DIM), w_att[l].reshape(1, DIM))
        outs.append(o_l)
        atts.append(a_l)

    feat, misc = _pool_call(outs[0], outs[1], outs[2],
                            atts[0], atts[1], atts[2], h, batch_pad)
    return misc[:, 0], h[:N], feat


# speedup vs baseline: 3.0126x; 1.0488x over previous
import functools

import jax
import jax.numpy as jnp
from jax import lax
from jax.experimental import pallas as pl
from jax.experimental.pallas import tpu as pltpu
from jax.experimental.pallas import tpu_sc as plsc

N = 10000
E = 320000
DIM = 128
NUM_RBF = 16
N_LAYER = 3
CUTOFF = 5.0
NUM_GRAPHS = 512

NC = 2
NS = 16
NW = NC * NS
C = 128
NCH = 80
E_PAD = NW * NCH * C
ROWS_PER_TILE = 640
AGG_ROWS = NS * ROWS_PER_TILE
POS_PAD = 30720

NP = 10240
NB = 2048
EB = 4096


def _dist_body(pos_hbm, j_hbm, i_hbm, d2_hbm, posv, jv, iv, d2v):
    w = lax.axis_index("s") * NC + lax.axis_index("c")
    pltpu.sync_copy(pos_hbm, posv)
    pltpu.sync_copy(j_hbm.at[pl.ds(w * NCH, NCH)], jv)
    pltpu.sync_copy(i_hbm.at[pl.ds(w * NCH, NCH)], iv)

    def chunk(c, _):
        def vec(v, _):
            sl = pl.ds(v * 16, 16)
            jj = jv[c, sl] * 3
            ii = iv[c, sl] * 3
            xj = plsc.load_gather(posv, [jj])
            yj = plsc.load_gather(posv, [jj + 1])
            zj = plsc.load_gather(posv, [jj + 2])
            xi = plsc.load_gather(posv, [ii])
            yi = plsc.load_gather(posv, [ii + 1])
            zi = plsc.load_gather(posv, [ii + 2])
            dx = xi - xj
            dy = yi - yj
            dz = zi - zj
            d2v[c, sl] = dx * dx + dy * dy + dz * dz
            return 0

        lax.fori_loop(0, C // 16, vec, 0)
        return 0

    lax.fori_loop(0, NCH, chunk, 0)
    pltpu.sync_copy(d2v, d2_hbm.at[pl.ds(w * NCH, NCH)])


def _dist_call(pos_flat, j2, i2):
    mesh = plsc.VectorSubcoreMesh(core_axis_name="c", subcore_axis_name="s")
    return pl.kernel(
        _dist_body,
        out_type=jax.ShapeDtypeStruct((NW * NCH, C), jnp.float32),
        mesh=mesh,
        compiler_params=pltpu.CompilerParams(needs_layout_passes=False),
        scratch_types=[
            pltpu.VMEM((POS_PAD,), jnp.float32),
            pltpu.VMEM((NCH, C), jnp.int32),
            pltpu.VMEM((NCH, C), jnp.int32),
            pltpu.VMEM((NCH, C), jnp.float32),
        ],
    )(pos_flat, j2, i2)


CB = 64
NCH2 = E_PAD // (NW * CB)
GRP = 32
NGRP = NCH2 // GRP


def _agg_body(hw_hbm, ea_hbm, j_hbm, i_hbm, out_hbm,
              jv, iv, r0, r1, e0, e1, aggs, sg0, sg1, se0, se1, ss0, ss1):
    c_id = lax.axis_index("c")
    s_id = lax.axis_index("s")
    w = s_id * NC + c_id
    rbufs = (r0, r1)
    ebufs = (e0, e1)
    sgs = (sg0, sg1)
    ses = (se0, se1)
    sss = (ss0, ss1)

    def zrow(r, _):
        for d in range(8):
            r0[r, pl.ds(d * 16, 16)] = jnp.zeros((16,), jnp.float32)
        return 0

    lax.fori_loop(0, CB, zrow, 0)
    for z in range(ROWS_PER_TILE // CB):
        pltpu.sync_copy(r0, aggs.at[pl.ds(s_id * ROWS_PER_TILE + z * CB, CB)])
    plsc.subcore_barrier()

    def group(g, _):
        gbase = w * NCH2 + g * GRP
        pltpu.sync_copy(j_hbm.at[pl.ds(gbase, GRP)], jv)
        pltpu.sync_copy(i_hbm.at[pl.ds(gbase, GRP)], iv)
        d_g = [None, None]
        d_e = [None, None]
        d_s = [None, None]
        d_g[0] = pltpu.async_copy(hw_hbm.at[jv.at[0]], r0, sg0)
        d_e[0] = pltpu.async_copy(ea_hbm.at[pl.ds(gbase * CB, CB)], e0, se0)
        for k in range(GRP):
            kb = k % 2
            nb = (k + 1) % 2
            if k < GRP - 1:
                if k >= 1:
                    d_s[nb].wait()
                d_g[nb] = pltpu.async_copy(hw_hbm.at[jv.at[k + 1]], rbufs[nb], sgs[nb])
                d_e[nb] = pltpu.async_copy(
                    ea_hbm.at[pl.ds((gbase + k + 1) * CB, CB)], ebufs[nb], ses[nb])
            d_g[kb].wait()
            d_e[kb].wait()
            rows = rbufs[kb]
            eav = ebufs[kb]

            def mulrow(r, _):
                for d in range(8):
                    sl = pl.ds(d * 16, 16)
                    rows[r, sl] = rows[r, sl] * eav[r, sl]
                return 0

            lax.fori_loop(0, CB, mulrow, 0)
            d_s[kb] = pltpu.async_copy(rows, aggs.at[iv.at[k]], sss[kb], add=True)
        d_s[0].wait()
        d_s[1].wait()
        return 0

    lax.fori_loop(0, NGRP, group, 0)
    plsc.subcore_barrier()
    pltpu.sync_copy(
        aggs.at[pl.ds(s_id * ROWS_PER_TILE, ROWS_PER_TILE)],
        out_hbm.at[c_id, pl.ds(s_id * ROWS_PER_TILE, ROWS_PER_TILE)],
    )


def _agg_call(hw, ea, j64, i64):
    mesh = plsc.VectorSubcoreMesh(core_axis_name="c", subcore_axis_name="s")
    return pl.kernel(
        _agg_body,
        out_type=jax.ShapeDtypeStruct((NC, AGG_ROWS, DIM), jnp.float32),
        mesh=mesh,
        compiler_params=pltpu.CompilerParams(needs_layout_passes=False),
        scratch_types=[
            pltpu.VMEM((GRP, CB), jnp.int32),
            pltpu.VMEM((GRP, CB), jnp.int32),
            pltpu.VMEM((CB, DIM), jnp.float32),
            pltpu.VMEM((CB, DIM), jnp.float32),
            pltpu.VMEM((CB, DIM), jnp.float32),
            pltpu.VMEM((CB, DIM), jnp.float32),
            pltpu.VMEM_SHARED((AGG_ROWS, DIM), jnp.float32),
        ] + [pltpu.SemaphoreType.DMA] * 6,
    )(hw, ea, j64, i64)


def _embed_body(x_ref, emb_ref, wmsg_ref, h_ref, hw_ref):
    xb = x_ref[...]
    onehot = (xb[:, None] == lax.broadcasted_iota(jnp.int32, (NB, 32), 1))
    h = jnp.dot(onehot.astype(jnp.float32), emb_ref[...],
                preferred_element_type=jnp.float32)
    h_ref[...] = h
    hw_ref[...] = jnp.dot(h, wmsg_ref[...], preferred_element_type=jnp.float32)


def _embed_call(x, emb_pad, wmsg0):
    return pl.pallas_call(
        _embed_body,
        grid=(NP // NB,),
        in_specs=[
            pl.BlockSpec((NB,), lambda b: (b,)),
            pl.BlockSpec((32, DIM), lambda b: (0, 0)),
            pl.BlockSpec((DIM, DIM), lambda b: (0, 0)),
        ],
        out_specs=[
            pl.BlockSpec((NB, DIM), lambda b: (b, 0)),
            pl.BlockSpec((NB, DIM), lambda b: (b, 0)),
        ],
        out_shape=[
            jax.ShapeDtypeStruct((NP, DIM), jnp.float32),
            jax.ShapeDtypeStruct((NP, DIM), jnp.float32),
        ],
    )(x, emb_pad, wmsg0)


def _ea_body(d2_ref, npad_ref, wrbf_ref, ea_ref):
    d2 = d2_ref[...]
    dist = jnp.sqrt(d2 + 1e-12)
    npad = npad_ref[...]
    arg = npad * (jnp.pi / CUTOFF) * dist[:, None]
    rbf = jnp.sin(arg) / (dist[:, None] + 1e-6)
    env = jnp.where(dist < CUTOFF, (1.0 - dist / CUTOFF) ** 2, 0.0)
    rbf = rbf * env[:, None]
    ea_ref[...] = jnp.dot(rbf, wrbf_ref[...], preferred_element_type=jnp.float32)


def _ea_call(d2_flat, npad, wrbf_pad):
    return pl.pallas_call(
        _ea_body,
        grid=(E_PAD // EB,),
        in_specs=[
            pl.BlockSpec((EB,), lambda b: (b,)),
            pl.BlockSpec((1, DIM), lambda b: (0, 0)),
            pl.BlockSpec((DIM, DIM), lambda b: (0, 0)),
        ],
        out_specs=pl.BlockSpec((EB, DIM), lambda b: (b, 0)),
        out_shape=jax.ShapeDtypeStruct((E_PAD, DIM), jnp.float32),
    )(d2_flat, npad, wrbf_pad)


def _upd_body(h_ref, a0_ref, a1_ref, wupd_ref, wmsgn_ref, wout_ref, watt_ref,
              hn_ref, hw_ref, o_ref, a_ref):
    agg = a0_ref[...] + a1_ref[...]
    u = jnp.dot(agg, wupd_ref[...], preferred_element_type=jnp.float32)
    hn = h_ref[...] + u * jax.nn.sigmoid(u)
    hn_ref[...] = hn
    hw_ref[...] = jnp.dot(hn, wmsgn_ref[...], preferred_element_type=jnp.float32)
    o_ref[...] = jnp.sum(hn * wout_ref[...], axis=1)
    a_ref[...] = jnp.sum(hn * watt_ref[...], axis=1)


def _upd_call(h, a0, a1, wupd, wmsgn, wout_row, watt_row):
    return pl.pallas_call(
        _upd_body,
        grid=(NP // NB,),
        in_specs=[
            pl.BlockSpec((NB, DIM), lambda b: (b, 0)),
            pl.BlockSpec((NB, DIM), lambda b: (b, 0)),
            pl.BlockSpec((NB, DIM), lambda b: (b, 0)),
            pl.BlockSpec((DIM, DIM), lambda b: (0, 0)),
            pl.BlockSpec((DIM, DIM), lambda b: (0, 0)),
            pl.BlockSpec((1, DIM), lambda b: (0, 0)),
            pl.BlockSpec((1, DIM), lambda b: (0, 0)),
        ],
        out_specs=[
            pl.BlockSpec((NB, DIM), lambda b: (b, 0)),
            pl.BlockSpec((NB, DIM), lambda b: (b, 0)),
            pl.BlockSpec((NB,), lambda b: (b,)),
            pl.BlockSpec((NB,), lambda b: (b,)),
        ],
        out_shape=[
            jax.ShapeDtypeStruct((NP, DIM), jnp.float32),
            jax.ShapeDtypeStruct((NP, DIM), jnp.float32),
            jax.ShapeDtypeStruct((NP,), jnp.float32),
            jax.ShapeDtypeStruct((NP,), jnp.float32),
        ],
    )(h, a0, a1, wupd, wmsgn, wout_row, watt_row)


def _pool_body(o0_ref, o1_ref, o2_ref, a0_ref, a1_ref, a2_ref, h_ref, b_ref,
               feat_ref, misc_ref, feat_acc, misc_acc):
    b = pl.program_id(0)

    @pl.when(b == 0)
    def _():
        feat_acc[...] = jnp.zeros((NUM_GRAPHS, DIM), jnp.float32)
        misc_acc[...] = jnp.zeros((NUM_GRAPHS, DIM), jnp.float32)

    def leaky(v):
        return jnp.where(v > 0, v, 0.2 * v)

    l0 = leaky(a0_ref[...])
    l1 = leaky(a1_ref[...])
    l2 = leaky(a2_ref[...])
    m = jnp.maximum(jnp.maximum(l0, l1), l2)
    e0 = jnp.exp(l0 - m)
    e1 = jnp.exp(l1 - m)
    e2 = jnp.exp(l2 - m)
    s = e0 + e1 + e2
    energy = (o0_ref[...] * e0 + o1_ref[...] * e1 + o2_ref[...] * e2) / s

    batch = b_ref[...]
    onehot = (lax.broadcasted_iota(jnp.int32, (NUM_GRAPHS, NB), 0)
              == batch[None, :]).astype(jnp.float32)
    feat_acc[...] += jnp.dot(onehot, h_ref[...], preferred_element_type=jnp.float32)
    lane = lax.broadcasted_iota(jnp.int32, (NB, DIM), 1)
    eo = jnp.where(lane == 0, energy[:, None], 0.0) + jnp.where(lane == 1, 1.0, 0.0)
    misc_acc[...] += jnp.dot(onehot, eo, preferred_element_type=jnp.float32)

    @pl.when(b == NP // NB - 1)
    def _():
        misc = misc_acc[...]
        lane2 = lax.broadcasted_iota(jnp.int32, (NUM_GRAPHS, DIM), 1)
        counts = jnp.sum(jnp.where(lane2 == 1, misc, 0.0), axis=1, keepdims=True)
        feat_ref[...] = feat_acc[...] / jnp.maximum(counts, 1.0)
        misc_ref[...] = misc


def _pool_call(o0, o1, o2, a0, a1, a2, h, batch):
    return pl.pallas_call(
        _pool_body,
        grid=(NP // NB,),
        in_specs=[pl.BlockSpec((NB,), lambda b: (b,))] * 6
        + [
            pl.BlockSpec((NB, DIM), lambda b: (b, 0)),
            pl.BlockSpec((NB,), lambda b: (b,)),
        ],
        out_specs=[
            pl.BlockSpec((NUM_GRAPHS, DIM), lambda b: (0, 0)),
            pl.BlockSpec((NUM_GRAPHS, DIM), lambda b: (0, 0)),
        ],
        out_shape=[
            jax.ShapeDtypeStruct((NUM_GRAPHS, DIM), jnp.float32),
            jax.ShapeDtypeStruct((NUM_GRAPHS, DIM), jnp.float32),
        ],
        scratch_shapes=[
            pltpu.VMEM((NUM_GRAPHS, DIM), jnp.float32),
            pltpu.VMEM((NUM_GRAPHS, DIM), jnp.float32),
        ],
    )(o0, o1, o2, a0, a1, a2, h, batch)


def kernel(x, pos, edge_index, batch, embeddings, w_rbf, w_msg, w_upd, w_att, w_out):
    pad = E_PAD - E
    j = jnp.concatenate([edge_index[0], jnp.zeros((pad,), jnp.int32)])
    i = jnp.concatenate([edge_index[1], jnp.full((pad,), N, jnp.int32)])
    j2 = j.reshape(NW * NCH, C)
    i2 = i.reshape(NW * NCH, C)
    j64 = j.reshape(E_PAD // CB, CB)
    i64 = i.reshape(E_PAD // CB, CB)
    pos_flat = jnp.concatenate(
        [pos.reshape(-1), jnp.zeros((POS_PAD - 3 * N,), jnp.float32)])
    emb_pad = jnp.pad(embeddings, ((0, 32 - embeddings.shape[0]), (0, 0)))
    npad = jnp.pad(jnp.arange(1, NUM_RBF + 1, dtype=jnp.float32),
                   (0, DIM - NUM_RBF)).reshape(1, DIM)
    wrbf_pad = jnp.pad(w_rbf, ((0, DIM - NUM_RBF), (0, 0)))

    x_pad = jnp.pad(x, (0, NP - N))
    batch_pad = jnp.pad(batch, (0, NP - N), constant_values=NUM_GRAPHS)
    h, hw = _embed_call(x_pad, emb_pad, w_msg[0])
    d2 = _dist_call(pos_flat, j2, i2)
    ea = _ea_call(d2.reshape(-1), npad, wrbf_pad)

    outs, atts = [], []
    for l in range(N_LAYER):
        agg2 = _agg_call(hw, ea, j64, i64)
        wmsgn = w_msg[l + 1] if l + 1 < N_LAYER else w_msg[0]
        h, hw, o_l, a_l = _upd_call(
            h, agg2[0], agg2[1], w_upd[l], wmsgn,
            w_out[l].reshape(1, DIM), w_att[l].reshape(1, DIM))
        outs.append(o_l)
        atts.append(a_l)

    feat, misc = _pool_call(outs[0], outs[1], outs[2],
                            atts[0], atts[1], atts[2], h, batch_pad)
    return misc[:, 0], h[:N], feat
